# Initial kernel scaffold; baseline (speedup 1.0000x reference)
import jax
import jax.numpy as jnp
from jax.experimental import pallas as pl


def kernel(output):
    raise NotImplementedError("write your pallas kernel here")


# speedup vs baseline: 354.9249x; 354.9249x over previous
import jax
import jax.numpy as jnp
from jax.experimental import pallas as pl
from jax.experimental.pallas import tpu as pltpu

_NUM_ANCHORS = 5
_ANCHORS = [0.57273, 0.677385, 1.87446, 2.06253, 3.33843, 5.47434,
            7.88282, 3.52778, 9.77052, 9.16828]
_CONF_THRES = 0.6
_IOU_THRES = 0.4
_H = 19
_W = 19
_HW = _H * _W
_N = _NUM_ANCHORS * _HW
_NPAD = 1920
_BLK = 128
_NBLK = _NPAD // _BLK


def _sigmoid(x):
    return 1.0 / (1.0 + jnp.exp(-x))


def _decode(rx, ry, rw, rh, rc, idx):
    f = jnp.float32
    cell = idx % _HW
    a = idx // _HW
    gx = (cell % _W).astype(f)
    gy = (cell // _W).astype(f)
    aw = jnp.full_like(gx, _ANCHORS[8])
    ah = jnp.full_like(gx, _ANCHORS[9])
    for k in range(_NUM_ANCHORS - 1):
        aw = jnp.where(a == k, _ANCHORS[2 * k], aw)
        ah = jnp.where(a == k, _ANCHORS[2 * k + 1], ah)
    x = (_sigmoid(rx) + gx) / _W
    y = (_sigmoid(ry) + gy) / _H
    w = jnp.exp(rw) * aw / _W
    h = jnp.exp(rh) * ah / _H
    det = _sigmoid(rc)
    conf = jnp.where((det > _CONF_THRES) & (idx < _N), det, 0.0)
    hw_ = w * 0.5
    hh = h * 0.5
    return x - hw_, x + hw_, y - hh, y + hh, w * h, conf


def _nms_kernel(row_ref, xc_ref, yc_ref, wc_ref, hc_ref, cc_ref,
                out_ref, m_ref):
    f = jnp.float32
    jidx = jax.lax.broadcasted_iota(jnp.int32, (1, _NPAD), 1)
    x1j, x2j, y1j, y2j, areaj, cj = _decode(
        row_ref[0:1, :], row_ref[1:2, :], row_ref[2:3, :],
        row_ref[3:4, :], row_ref[4:5, :], jidx)

    def build_block(k, _):
        base = pl.multiple_of(k * _BLK, _BLK)
        iidx = base + jax.lax.broadcasted_iota(jnp.int32, (_BLK, 1), 0)
        x1i, x2i, y1i, y2i, areai, ci = _decode(
            xc_ref[pl.ds(base, _BLK), :], yc_ref[pl.ds(base, _BLK), :],
            wc_ref[pl.ds(base, _BLK), :], hc_ref[pl.ds(base, _BLK), :],
            cc_ref[pl.ds(base, _BLK), :], iidx)
        uw = jnp.maximum(x2i, x2j) - jnp.minimum(x1i, x1j)
        uh = jnp.maximum(y2i, y2j) - jnp.minimum(y1i, y1j)
        cw = (x2i - x1i) + (x2j - x1j) - uw
        ch = (y2i - y1i) + (y2j - y1j) - uh
        carea = cw * ch
        uarea = areai + areaj - carea
        overlap = (cw > 0) & (ch > 0) & (carea > _IOU_THRES * uarea)
        prec = (ci > cj) | ((ci == cj) & (iidx < jidx))
        m = overlap & prec & (ci > 0) & (cj > 0)
        m_ref[pl.ds(base, _BLK), :] = m.astype(jnp.bfloat16)
        return 0

    jax.lax.fori_loop(0, _NBLK, build_block, 0, unroll=False)

    tj = cj > 0
    alive0 = jnp.broadcast_to(tj.astype(f), (8, _NPAD))

    def cond(carry):
        _, changed, it = carry
        return changed & (it < _NPAD)

    def body(carry):
        alive, _, it = carry
        s = jax.lax.dot_general(
            alive.astype(jnp.bfloat16), m_ref[...],
            dimension_numbers=(((1,), (0,)), ((), ())),
            preferred_element_type=f)
        new = ((s == 0.0) & tj).astype(f)
        return new, jnp.any(new != alive), it + 1

    alive, _, _ = jax.lax.while_loop(cond, body, (alive0, True, 0))

    keep = alive[0:1, :]
    cnt = jnp.sum(keep, axis=1, keepdims=True)
    total = jnp.sum(cj * keep, axis=1, keepdims=True)
    out_ref[...] = jnp.where(cnt > 0, total / jnp.where(cnt > 0, cnt, 1.0), 0.0)


def kernel(output):
    raw = output[0].reshape(_NUM_ANCHORS, 5 + 80, _HW)[:, :5, :]
    rows = raw.transpose(1, 0, 2).reshape(5, _N)
    rows = jnp.pad(rows, ((0, 0), (0, _NPAD - _N)))
    cols = [rows[k].reshape(_NPAD, 1) for k in range(5)]
    res = pl.pallas_call(
        _nms_kernel,
        out_shape=jax.ShapeDtypeStruct((1, 1), jnp.float32),
        in_specs=[
            pl.BlockSpec((5, _NPAD), lambda: (0, 0)),
            pl.BlockSpec((_NPAD, 1), lambda: (0, 0)),
            pl.BlockSpec((_NPAD, 1), lambda: (0, 0)),
            pl.BlockSpec((_NPAD, 1), lambda: (0, 0)),
            pl.BlockSpec((_NPAD, 1), lambda: (0, 0)),
            pl.BlockSpec((_NPAD, 1), lambda: (0, 0)),
        ],
        out_specs=pl.BlockSpec((1, 1), lambda: (0, 0)),
        scratch_shapes=[pltpu.VMEM((_NPAD, _NPAD), jnp.bfloat16)],
        compiler_params=pltpu.CompilerParams(
            vmem_limit_bytes=100 * 1024 * 1024),
    )(rows, *cols)
    return res.reshape(())
